# baseline (speedup 1.0000x reference)
import functools

import jax
import jax.numpy as jnp
from jax import lax
from jax.experimental import pallas as pl
from jax.experimental.pallas import tpu as pltpu
from jax.experimental.pallas import tpu_sc as plsc

N_NODE = 50000
N_PAD = 51200
EMB = 100
EMBP = 128
NCH = 4
W = 32
NNZ = 800000
NNZ_PAD = 802816
NNZ_TILE = 50176
N_ITER = 98
BATCH = 1024
SEQ = 50
ROWS_TILE = N_PAD // 16
F32 = jnp.float32


def _spmm_body(scale, write_y, *refs):
    if write_y:
        (xflat, coladj, rowr, valp, prevflat, yflat, combflat,
         acc, zbuf, col_a, row_a, val_a, rows_a, sem_a,
         col_b, row_b, val_b, rows_b, sem_b, dbuf, pbuf) = refs
    else:
        (xflat, coladj, rowr, valp, prevflat, combflat,
         acc, zbuf, col_a, row_a, val_a, rows_a, sem_a,
         col_b, row_b, val_b, rows_b, sem_b, dbuf, pbuf) = refs
        yflat = None
    bufs = ((col_a, row_a, val_a, rows_a, sem_a),
            (col_b, row_b, val_b, rows_b, sem_b))
    c = lax.axis_index("c")
    s = lax.axis_index("s")

    zero16 = jnp.zeros((16,), F32)

    def _zb(r, carry):
        zbuf[r, pl.ds(0, 16)] = zero16
        zbuf[r, pl.ds(16, 16)] = zero16
        return carry
    lax.fori_loop(0, 128, _zb, 0)

    nb = s * (NNZ_TILE // 128)
    vb = s * NNZ_TILE
    zrow0 = s * ROWS_TILE

    for p in range(2):
        k = 2 * c + p

        def _zloop(j, carry):
            pltpu.sync_copy(zbuf, acc.at[pl.ds(zrow0 + j * 64, 64)])
            return carry
        lax.fori_loop(0, 50, _zloop, 0)
        plsc.subcore_barrier()

        def _load_fire(gb, bset):
            colv, rowv, valv, rowsv, sm = bset
            pltpu.sync_copy(coladj.at[k, pl.ds(nb + gb * 2, 2), :], colv)
            pltpu.sync_copy(rowr.at[pl.ds(nb + gb * 2, 2), :], rowv)
            pltpu.sync_copy(valp.at[pl.ds(vb + gb * 256, 256)], valv)
            for j in range(2):
                pltpu.async_copy(xflat.at[colv.at[j]],
                                 rowsv.at[pl.ds(j * 128, 128)], sm)

        _load_fire(0, bufs[0])

        def _gbody(g, carry):
            for b in range(2):
                gb = g * 2 + b
                colv, rowv, valv, rowsv, sm = bufs[b]

                @pl.when(gb + 1 < 2 * N_ITER)
                def _():
                    _load_fire(gb + 1, bufs[1 - b])

                for j in range(2):
                    pltpu.make_async_copy(
                        xflat.at[colv.at[j]],
                        rowsv.at[pl.ds(j * 128, 128)], sm).wait()

                def _mulq(q, carry2):
                    vals16 = valv[pl.ds(q * 16, 16)]
                    for i in range(16):
                        r = q * 16 + i
                        v = vals16.at[jnp.full((16,), i, jnp.int32)] \
                                  .get(mode="promise_in_bounds")
                        rowsv[r, pl.ds(0, 16)] = rowsv[r, pl.ds(0, 16)] * v
                        rowsv[r, pl.ds(16, 16)] = \
                            rowsv[r, pl.ds(16, 16)] * v
                    return carry2
                lax.fori_loop(0, 16, _mulq, 0)

                for j in range(2):
                    pltpu.sync_copy(rowsv.at[pl.ds(j * 128, 128)],
                                    acc.at[rowv.at[j]], add=True)
            return carry
        lax.fori_loop(0, N_ITER, _gbody, 0)
        plsc.subcore_barrier()

        def _dloop(j, carry):
            pltpu.sync_copy(acc.at[pl.ds(zrow0 + j * 64, 64)], dbuf)
            pltpu.sync_copy(
                prevflat.at[pl.ds(k * N_PAD + zrow0 + j * 64, 64)],
                pbuf)

            def _cb(q, carry2):
                for i in range(8):
                    r = q * 8 + i
                    a0 = (dbuf[r, pl.ds(0, 16)]
                          + pbuf[r, pl.ds(0, 16)]) * scale
                    a1 = (dbuf[r, pl.ds(16, 16)]
                          + pbuf[r, pl.ds(16, 16)]) * scale
                    pbuf[r, pl.ds(0, 16)] = a0
                    pbuf[r, pl.ds(16, 16)] = a1
                return carry2
            lax.fori_loop(0, 8, _cb, 0)
            if yflat is not None:
                pltpu.sync_copy(
                    dbuf,
                    yflat.at[pl.ds(k * N_PAD + zrow0 + j * 64, 64)])
            pltpu.sync_copy(
                pbuf,
                combflat.at[pl.ds(k * N_PAD + zrow0 + j * 64, 64)])
            return carry
        lax.fori_loop(0, 50, _dloop, 0)
        plsc.subcore_barrier()


@functools.lru_cache(maxsize=None)
def _make_spmm(scale, write_y):
    mesh = plsc.VectorSubcoreMesh(core_axis_name="c", subcore_axis_name="s",
                                  num_cores=2, num_subcores=16)
    outs = []
    if write_y:
        outs.append(jax.ShapeDtypeStruct((NCH * N_PAD, W), F32))
    outs.append(jax.ShapeDtypeStruct((NCH * N_PAD, W), F32))
    return pl.kernel(
        functools.partial(_spmm_body, scale, write_y),
        out_type=outs,
        mesh=mesh,
        compiler_params=pltpu.CompilerParams(use_tc_tiling_on_sc=False),
        scratch_types=[
            pltpu.VMEM_SHARED((N_PAD, W), F32),
            pltpu.VMEM((64, W), F32),
            pltpu.VMEM((2, 128), jnp.int32),
            pltpu.VMEM((2, 128), jnp.int32),
            pltpu.VMEM((256,), F32),
            pltpu.VMEM((256, W), F32),
            pltpu.SemaphoreType.DMA,
            pltpu.VMEM((2, 128), jnp.int32),
            pltpu.VMEM((2, 128), jnp.int32),
            pltpu.VMEM((256,), F32),
            pltpu.VMEM((256, W), F32),
            pltpu.SemaphoreType.DMA,
            pltpu.VMEM((64, W), F32),
            pltpu.VMEM((64, W), F32),
        ],
    )


def _gather_body(table, ridx, out, idx_v, rows_v, sem):
    c = lax.axis_index("c")
    s = lax.axis_index("s")
    w = s * 2 + c
    for i in range(13):
        ch = w + i * 32

        @pl.when(ch < 400)
        def _():
            pltpu.sync_copy(ridx.at[ch], idx_v)
            pltpu.async_copy(table.at[idx_v], rows_v, sem).wait()
            pltpu.sync_copy(rows_v, out.at[pl.ds(ch * 128, 128)])


@functools.lru_cache(maxsize=None)
def _make_gather():
    mesh = plsc.VectorSubcoreMesh(core_axis_name="c", subcore_axis_name="s",
                                  num_cores=2, num_subcores=16)
    return pl.kernel(
        _gather_body,
        out_type=[jax.ShapeDtypeStruct((BATCH * SEQ, EMBP), F32)],
        mesh=mesh,
        scratch_types=[
            pltpu.VMEM((128,), jnp.int32),
            pltpu.VMEM((128, EMBP), F32),
            pltpu.SemaphoreType.DMA,
        ],
    )


BB = 128


def _attn_body(seq_ref, zm_ref, mf_ref, sl_ref, pos_ref, w1a_ref, w1b_ref,
               b1_ref, g1W_ref, g1b_ref, g2W_ref, w2_ref, out_ref):
    prec = lax.Precision.HIGHEST
    sh = seq_ref[...] * zm_ref[...][:, :, None]
    hs = jnp.sum(sh, axis=1) / sl_ref[...]
    posp = jnp.dot(pos_ref[...], w1a_ref[...],
                   preferred_element_type=F32, precision=prec)
    t = jnp.dot(sh.reshape(BB * SEQ, EMBP), w1b_ref[...],
                preferred_element_type=F32, precision=prec)
    nh = jnp.tanh(t.reshape(BB, SEQ, EMB) + posp[None] + b1_ref[...])
    g = jnp.dot(nh.reshape(BB * SEQ, EMB), g1W_ref[...],
                preferred_element_type=F32, precision=prec).reshape(BB, SEQ, EMB)
    h2 = jnp.dot(hs, g2W_ref[...],
                 preferred_element_type=F32, precision=prec)
    nh2 = jax.nn.sigmoid(g + g1b_ref[...] + h2[:, None, :])
    beta = jnp.sum(nh2 * w2_ref[...], axis=-1, keepdims=True)
    beta = beta * mf_ref[...][:, :, None]
    sel = jnp.sum(beta * sh, axis=1)
    out_ref[...] = sel[:, :EMB]


def _attn(seqh3, zmask, maskf, slen, pos50, w1a, w1b, b1, g1W, g1b, g2W, w2r):
    grid = (BATCH // BB,)
    full = lambda shape: pl.BlockSpec(shape, lambda b: (0,) * len(shape))
    return pl.pallas_call(
        _attn_body,
        grid=grid,
        in_specs=[
            pl.BlockSpec((BB, SEQ, EMBP), lambda b: (b, 0, 0)),
            pl.BlockSpec((BB, SEQ), lambda b: (b, 0)),
            pl.BlockSpec((BB, SEQ), lambda b: (b, 0)),
            pl.BlockSpec((BB, 1), lambda b: (b, 0)),
            full((SEQ, EMB)),
            full((EMB, EMB)),
            full((EMBP, EMB)),
            full((1, EMB)),
            full((EMB, EMB)),
            full((1, EMB)),
            full((EMBP, EMB)),
            full((1, EMB)),
        ],
        out_specs=pl.BlockSpec((BB, EMB), lambda b: (b, 0)),
        out_shape=jax.ShapeDtypeStruct((BATCH, EMB), F32),
    )(seqh3, zmask, maskf, slen, pos50, w1a, w1b, b1, g1W, g1b, g2W, w2r)


def kernel(embedding, pos_embedding, w1_W, w1_b, w_2, glu1_W, glu1_b, glu2_W,
           adj_val, session_len, adj_idx, session_item, reversed_sess_item,
           mask):
    row = adj_idx[0].astype(jnp.int32)
    col = adj_idx[1].astype(jnp.int32)
    pad_n = NNZ_PAD - NNZ
    fill = (jnp.arange(pad_n, dtype=jnp.int32) * 977) % N_NODE
    colp = jnp.concatenate([col, fill])
    rowp = jnp.concatenate([row, fill])
    valp = jnp.concatenate([adj_val, jnp.zeros((pad_n,), F32)])
    coladj = (colp[None, :]
              + (jnp.arange(NCH, dtype=jnp.int32) * N_PAD)[:, None]
              ).reshape(NCH, NNZ_PAD // 128, 128)
    rowr = rowp.reshape(NNZ_PAD // 128, 128)

    emb_pad = jnp.pad(embedding, ((0, N_PAD - N_NODE), (0, EMBP - EMB)))
    embflat = emb_pad.reshape(N_PAD, NCH, W).transpose(1, 0, 2) \
                     .reshape(NCH * N_PAD, W)

    y1, c1 = _make_spmm(1.0, True)(embflat, coladj, rowr, valp, embflat)
    comb2, = _make_spmm(1.0 / 3.0, False)(y1, coladj, rowr, valp, c1)
    tablepad = comb2.reshape(NCH, N_PAD, W).transpose(1, 0, 2) \
                    .reshape(N_PAD, EMBP)
    item_hg = tablepad[:N_NODE, :EMB]

    ridx = jnp.maximum(reversed_sess_item.astype(jnp.int32) - 1, 0) \
              .reshape(BATCH * SEQ // 128, 128)
    seqh, = _make_gather()(tablepad, ridx)
    seqh3 = seqh.reshape(BATCH, SEQ, EMBP)

    zmask = (reversed_sess_item != 0).astype(F32)
    maskf = mask.astype(F32)
    w1a = w1_W[:EMB]
    w1b = jnp.pad(w1_W[EMB:], ((0, EMBP - EMB), (0, 0)))
    g2W = jnp.pad(glu2_W, ((0, EMBP - EMB), (0, 0)))
    select = _attn(seqh3, zmask, maskf, session_len, pos_embedding[:SEQ],
                   w1a, w1b, w1_b.reshape(1, EMB), glu1_W,
                   glu1_b.reshape(1, EMB), g2W, w_2.reshape(1, EMB))
    return (item_hg, select)


# speedup vs baseline: 1.1588x; 1.1588x over previous
import functools

import jax
import jax.numpy as jnp
from jax import lax
from jax.experimental import pallas as pl
from jax.experimental.pallas import tpu as pltpu
from jax.experimental.pallas import tpu_sc as plsc

N_NODE = 50000
N_PAD = 51200
EMB = 100
EMBP = 128
NCH = 4
W = 32
NNZ = 800000
NNZ_PAD = 802816
NNZ_TILE = 50176
N_ITER = 98
BATCH = 1024
SEQ = 50
ROWS_TILE = N_PAD // 16
F32 = jnp.float32


def _spmm_body(scale, write_y, *refs):
    if write_y:
        (xflat, icr, valr, prevflat, yflat, combflat,
         acc, zbuf, slab_a, val_a, rows_a, sem_a,
         slab_b, val_b, rows_b, sem_b, dbuf, pbuf) = refs
    else:
        (xflat, icr, valr, prevflat, combflat,
         acc, zbuf, slab_a, val_a, rows_a, sem_a,
         slab_b, val_b, rows_b, sem_b, dbuf, pbuf) = refs
        yflat = None
    bufs = ((slab_a, val_a, rows_a, sem_a), (slab_b, val_b, rows_b, sem_b))
    c = lax.axis_index("c")
    s = lax.axis_index("s")

    zero16 = jnp.zeros((16,), F32)

    def _zb(r, carry):
        zbuf[r, pl.ds(0, 16)] = zero16
        zbuf[r, pl.ds(16, 16)] = zero16
        return carry
    lax.fori_loop(0, 128, _zb, 0)

    nb = s * (NNZ_TILE // 256)
    zrow0 = s * ROWS_TILE

    for p in range(2):
        k = 2 * c + p

        def _zloop(j, carry):
            pltpu.sync_copy(zbuf, acc.at[pl.ds(zrow0 + j * 64, 64)])
            return carry
        lax.fori_loop(0, 50, _zloop, 0)
        plsc.subcore_barrier()

        def _load_fire(gb, bset):
            slab, valv, rowsv, sm = bset
            pltpu.sync_copy(icr.at[k, nb + gb], slab)
            pltpu.sync_copy(valr.at[nb + gb], valv)
            for j in range(2):
                pltpu.async_copy(xflat.at[slab.at[j]],
                                 rowsv.at[pl.ds(j * 128, 128)], sm)

        _load_fire(0, bufs[0])

        def _gbody(g, carry):
            for b in range(2):
                gb = g * 2 + b
                slab, valv, rowsv, sm = bufs[b]

                @pl.when(gb + 1 < 2 * N_ITER)
                def _():
                    _load_fire(gb + 1, bufs[1 - b])

                for j in range(2):
                    pltpu.make_async_copy(
                        xflat.at[slab.at[j]],
                        rowsv.at[pl.ds(j * 128, 128)], sm).wait()

                def _mulq(q, carry2):
                    vals16 = valv[q // 8, pl.ds((q % 8) * 16, 16)]
                    for i in range(16):
                        r = q * 16 + i
                        v = vals16.at[jnp.full((16,), i, jnp.int32)] \
                                  .get(mode="promise_in_bounds")
                        rowsv[r, pl.ds(0, 16)] = rowsv[r, pl.ds(0, 16)] * v
                        rowsv[r, pl.ds(16, 16)] = \
                            rowsv[r, pl.ds(16, 16)] * v
                    return carry2
                lax.fori_loop(0, 16, _mulq, 0)

                for j in range(2):
                    pltpu.sync_copy(rowsv.at[pl.ds(j * 128, 128)],
                                    acc.at[slab.at[2 + j]], add=True)
            return carry
        lax.fori_loop(0, N_ITER, _gbody, 0)
        plsc.subcore_barrier()

        def _dloop(j, carry):
            pltpu.sync_copy(acc.at[pl.ds(zrow0 + j * 64, 64)], dbuf)
            pltpu.sync_copy(
                prevflat.at[pl.ds(k * N_PAD + zrow0 + j * 64, 64)],
                pbuf)

            def _cb(q, carry2):
                for i in range(8):
                    r = q * 8 + i
                    a0 = (dbuf[r, pl.ds(0, 16)]
                          + pbuf[r, pl.ds(0, 16)]) * scale
                    a1 = (dbuf[r, pl.ds(16, 16)]
                          + pbuf[r, pl.ds(16, 16)]) * scale
                    pbuf[r, pl.ds(0, 16)] = a0
                    pbuf[r, pl.ds(16, 16)] = a1
                return carry2
            lax.fori_loop(0, 8, _cb, 0)
            if yflat is not None:
                pltpu.sync_copy(
                    dbuf,
                    yflat.at[pl.ds(k * N_PAD + zrow0 + j * 64, 64)])
            pltpu.sync_copy(
                pbuf,
                combflat.at[pl.ds(k * N_PAD + zrow0 + j * 64, 64)])
            return carry
        lax.fori_loop(0, 50, _dloop, 0)
        plsc.subcore_barrier()


@functools.lru_cache(maxsize=None)
def _make_spmm(scale, write_y):
    mesh = plsc.VectorSubcoreMesh(core_axis_name="c", subcore_axis_name="s",
                                  num_cores=2, num_subcores=16)
    outs = []
    if write_y:
        outs.append(jax.ShapeDtypeStruct((NCH * N_PAD, W), F32))
    outs.append(jax.ShapeDtypeStruct((NCH * N_PAD, W), F32))
    return pl.kernel(
        functools.partial(_spmm_body, scale, write_y),
        out_type=outs,
        mesh=mesh,
        compiler_params=pltpu.CompilerParams(use_tc_tiling_on_sc=False),
        scratch_types=[
            pltpu.VMEM_SHARED((N_PAD, W), F32),
            pltpu.VMEM((64, W), F32),
            pltpu.VMEM((4, 128), jnp.int32),
            pltpu.VMEM((2, 128), F32),
            pltpu.VMEM((256, W), F32),
            pltpu.SemaphoreType.DMA,
            pltpu.VMEM((4, 128), jnp.int32),
            pltpu.VMEM((2, 128), F32),
            pltpu.VMEM((256, W), F32),
            pltpu.SemaphoreType.DMA,
            pltpu.VMEM((64, W), F32),
            pltpu.VMEM((64, W), F32),
        ],
    )


def _gather_body(table, ridx, out, idx_v, rows_v, sem):
    c = lax.axis_index("c")
    s = lax.axis_index("s")
    w = s * 2 + c
    for i in range(13):
        ch = w + i * 32

        @pl.when(ch < 400)
        def _():
            pltpu.sync_copy(ridx.at[ch], idx_v)
            pltpu.async_copy(table.at[idx_v], rows_v, sem).wait()
            pltpu.sync_copy(rows_v, out.at[pl.ds(ch * 128, 128)])


@functools.lru_cache(maxsize=None)
def _make_gather():
    mesh = plsc.VectorSubcoreMesh(core_axis_name="c", subcore_axis_name="s",
                                  num_cores=2, num_subcores=16)
    return pl.kernel(
        _gather_body,
        out_type=[jax.ShapeDtypeStruct((BATCH * SEQ, EMBP), F32)],
        mesh=mesh,
        scratch_types=[
            pltpu.VMEM((128,), jnp.int32),
            pltpu.VMEM((128, EMBP), F32),
            pltpu.SemaphoreType.DMA,
        ],
    )


BB = 128


def _attn_body(seq_ref, zm_ref, mf_ref, sl_ref, pos_ref, w1a_ref, w1b_ref,
               b1_ref, g1W_ref, g1b_ref, g2W_ref, w2_ref, out_ref):
    prec = lax.Precision.HIGHEST
    sh = seq_ref[...] * zm_ref[...][:, :, None]
    hs = jnp.sum(sh, axis=1) / sl_ref[...]
    posp = jnp.dot(pos_ref[...], w1a_ref[...],
                   preferred_element_type=F32, precision=prec)
    t = jnp.dot(sh.reshape(BB * SEQ, EMBP), w1b_ref[...],
                preferred_element_type=F32, precision=prec)
    nh = jnp.tanh(t.reshape(BB, SEQ, EMB) + posp[None] + b1_ref[...])
    g = jnp.dot(nh.reshape(BB * SEQ, EMB), g1W_ref[...],
                preferred_element_type=F32, precision=prec).reshape(BB, SEQ, EMB)
    h2 = jnp.dot(hs, g2W_ref[...],
                 preferred_element_type=F32, precision=prec)
    nh2 = jax.nn.sigmoid(g + g1b_ref[...] + h2[:, None, :])
    beta = jnp.sum(nh2 * w2_ref[...], axis=-1, keepdims=True)
    beta = beta * mf_ref[...][:, :, None]
    sel = jnp.sum(beta * sh, axis=1)
    out_ref[...] = sel[:, :EMB]


def _attn(seqh3, zmask, maskf, slen, pos50, w1a, w1b, b1, g1W, g1b, g2W, w2r):
    grid = (BATCH // BB,)
    full = lambda shape: pl.BlockSpec(shape, lambda b: (0,) * len(shape))
    return pl.pallas_call(
        _attn_body,
        grid=grid,
        in_specs=[
            pl.BlockSpec((BB, SEQ, EMBP), lambda b: (b, 0, 0)),
            pl.BlockSpec((BB, SEQ), lambda b: (b, 0)),
            pl.BlockSpec((BB, SEQ), lambda b: (b, 0)),
            pl.BlockSpec((BB, 1), lambda b: (b, 0)),
            full((SEQ, EMB)),
            full((EMB, EMB)),
            full((EMBP, EMB)),
            full((1, EMB)),
            full((EMB, EMB)),
            full((1, EMB)),
            full((EMBP, EMB)),
            full((1, EMB)),
        ],
        out_specs=pl.BlockSpec((BB, EMB), lambda b: (b, 0)),
        out_shape=jax.ShapeDtypeStruct((BATCH, EMB), F32),
    )(seqh3, zmask, maskf, slen, pos50, w1a, w1b, b1, g1W, g1b, g2W, w2r)


def kernel(embedding, pos_embedding, w1_W, w1_b, w_2, glu1_W, glu1_b, glu2_W,
           adj_val, session_len, adj_idx, session_item, reversed_sess_item,
           mask):
    row = adj_idx[0].astype(jnp.int32)
    col = adj_idx[1].astype(jnp.int32)
    pad_n = NNZ_PAD - NNZ
    fill = (jnp.arange(pad_n, dtype=jnp.int32) * 977) % N_NODE
    colp = jnp.concatenate([col, fill])
    rowp = jnp.concatenate([row, fill])
    valp = jnp.concatenate([adj_val, jnp.zeros((pad_n,), F32)])
    nbat = NNZ_PAD // 256
    coladj = (colp[None, :]
              + (jnp.arange(NCH, dtype=jnp.int32) * N_PAD)[:, None]
              ).reshape(NCH, nbat, 2, 128)
    rowb = jnp.broadcast_to(rowp.reshape(1, nbat, 2, 128),
                            (NCH, nbat, 2, 128))
    icr = jnp.concatenate([coladj, rowb], axis=2)
    valr = valp.reshape(nbat, 2, 128)

    emb_pad = jnp.pad(embedding, ((0, N_PAD - N_NODE), (0, EMBP - EMB)))
    embflat = emb_pad.reshape(N_PAD, NCH, W).transpose(1, 0, 2) \
                     .reshape(NCH * N_PAD, W)

    y1, c1 = _make_spmm(1.0, True)(embflat, icr, valr, embflat)
    comb2, = _make_spmm(1.0 / 3.0, False)(y1, icr, valr, c1)
    tablepad = comb2.reshape(NCH, N_PAD, W).transpose(1, 0, 2) \
                    .reshape(N_PAD, EMBP)
    item_hg = tablepad[:N_NODE, :EMB]

    ridx = jnp.maximum(reversed_sess_item.astype(jnp.int32) - 1, 0) \
              .reshape(BATCH * SEQ // 128, 128)
    seqh, = _make_gather()(tablepad, ridx)
    seqh3 = seqh.reshape(BATCH, SEQ, EMBP)

    zmask = (reversed_sess_item != 0).astype(F32)
    maskf = mask.astype(F32)
    w1a = w1_W[:EMB]
    w1b = jnp.pad(w1_W[EMB:], ((0, EMBP - EMB), (0, 0)))
    g2W = jnp.pad(glu2_W, ((0, EMBP - EMB), (0, 0)))
    select = _attn(seqh3, zmask, maskf, session_len, pos_embedding[:SEQ],
                   w1a, w1b, w1_b.reshape(1, EMB), glu1_W,
                   glu1_b.reshape(1, EMB), g2W, w_2.reshape(1, EMB))
    return (item_hg, select)


# speedup vs baseline: 1.1866x; 1.0239x over previous
import functools

import jax
import jax.numpy as jnp
from jax import lax
from jax.experimental import pallas as pl
from jax.experimental.pallas import tpu as pltpu
from jax.experimental.pallas import tpu_sc as plsc

N_NODE = 50000
N_PAD = 51200
EMB = 100
EMBP = 128
NCH = 4
W = 32
NNZ = 800000
NNZ_PAD = 802816
NNZ_TILE = 50176
N_ITER = 98
BATCH = 1024
SEQ = 50
ROWS_TILE = N_PAD // 16
F32 = jnp.float32


def _spmm_body(scale, write_y, *refs):
    if write_y:
        (xflat, icr, valr, prevflat, yflat, combflat,
         acc, zbuf, slab_a, val_a, rows_a, sem_a, ssem_a,
         slab_b, val_b, rows_b, sem_b, ssem_b, dbuf, pbuf) = refs
    else:
        (xflat, icr, valr, prevflat, combflat,
         acc, zbuf, slab_a, val_a, rows_a, sem_a, ssem_a,
         slab_b, val_b, rows_b, sem_b, ssem_b, dbuf, pbuf) = refs
        yflat = None
    bufs = ((slab_a, val_a, rows_a, sem_a, ssem_a),
            (slab_b, val_b, rows_b, sem_b, ssem_b))
    c = lax.axis_index("c")
    s = lax.axis_index("s")

    zero16 = jnp.zeros((16,), F32)

    def _zb(r, carry):
        zbuf[r, pl.ds(0, 16)] = zero16
        zbuf[r, pl.ds(16, 16)] = zero16
        return carry
    lax.fori_loop(0, 128, _zb, 0)

    nb = s * (NNZ_TILE // 256)
    zrow0 = s * ROWS_TILE

    for p in range(2):
        k = 2 * c + p

        def _zloop(j, carry):
            pltpu.sync_copy(zbuf, acc.at[pl.ds(zrow0 + j * 64, 64)])
            return carry
        lax.fori_loop(0, 50, _zloop, 0)
        plsc.subcore_barrier()

        def _load_fire(gb, bset):
            slab, valv, rowsv, sm, _ = bset
            pltpu.sync_copy(icr.at[k, nb + gb], slab)
            pltpu.sync_copy(valr.at[nb + gb], valv)
            for j in range(2):
                pltpu.async_copy(xflat.at[slab.at[j]],
                                 rowsv.at[pl.ds(j * 128, 128)], sm)

        def _wait_scatter(bset):
            slab, _, rowsv, _, ssm = bset
            for j in range(2):
                pltpu.make_async_copy(
                    rowsv.at[pl.ds(j * 128, 128)],
                    acc.at[slab.at[2 + j]], ssm).wait()

        _load_fire(0, bufs[0])

        def _gbody(g, carry):
            for b in range(2):
                gb = g * 2 + b
                slab, valv, rowsv, sm, ssm = bufs[b]

                @pl.when(gb >= 1)
                def _():
                    _wait_scatter(bufs[1 - b])

                @pl.when(gb + 1 < 2 * N_ITER)
                def _():
                    _load_fire(gb + 1, bufs[1 - b])

                for j in range(2):
                    pltpu.make_async_copy(
                        xflat.at[slab.at[j]],
                        rowsv.at[pl.ds(j * 128, 128)], sm).wait()

                def _mulq(q, carry2):
                    vals16 = valv[q // 8, pl.ds((q % 8) * 16, 16)]
                    for i in range(16):
                        r = q * 16 + i
                        v = vals16.at[jnp.full((16,), i, jnp.int32)] \
                                  .get(mode="promise_in_bounds")
                        rowsv[r, pl.ds(0, 16)] = rowsv[r, pl.ds(0, 16)] * v
                        rowsv[r, pl.ds(16, 16)] = \
                            rowsv[r, pl.ds(16, 16)] * v
                    return carry2
                lax.fori_loop(0, 16, _mulq, 0)

                for j in range(2):
                    pltpu.async_copy(rowsv.at[pl.ds(j * 128, 128)],
                                     acc.at[slab.at[2 + j]], ssm,
                                     add=True)
            return carry
        lax.fori_loop(0, N_ITER, _gbody, 0)
        _wait_scatter(bufs[1])
        plsc.subcore_barrier()

        def _dloop(j, carry):
            pltpu.sync_copy(acc.at[pl.ds(zrow0 + j * 64, 64)], dbuf)
            pltpu.sync_copy(
                prevflat.at[pl.ds(k * N_PAD + zrow0 + j * 64, 64)],
                pbuf)

            def _cb(q, carry2):
                for i in range(8):
                    r = q * 8 + i
                    a0 = (dbuf[r, pl.ds(0, 16)]
                          + pbuf[r, pl.ds(0, 16)]) * scale
                    a1 = (dbuf[r, pl.ds(16, 16)]
                          + pbuf[r, pl.ds(16, 16)]) * scale
                    pbuf[r, pl.ds(0, 16)] = a0
                    pbuf[r, pl.ds(16, 16)] = a1
                return carry2
            lax.fori_loop(0, 8, _cb, 0)
            if yflat is not None:
                pltpu.sync_copy(
                    dbuf,
                    yflat.at[pl.ds(k * N_PAD + zrow0 + j * 64, 64)])
            pltpu.sync_copy(
                pbuf,
                combflat.at[pl.ds(k * N_PAD + zrow0 + j * 64, 64)])
            return carry
        lax.fori_loop(0, 50, _dloop, 0)
        plsc.subcore_barrier()


@functools.lru_cache(maxsize=None)
def _make_spmm(scale, write_y):
    mesh = plsc.VectorSubcoreMesh(core_axis_name="c", subcore_axis_name="s",
                                  num_cores=2, num_subcores=16)
    outs = []
    if write_y:
        outs.append(jax.ShapeDtypeStruct((NCH * N_PAD, W), F32))
    outs.append(jax.ShapeDtypeStruct((NCH * N_PAD, W), F32))
    return pl.kernel(
        functools.partial(_spmm_body, scale, write_y),
        out_type=outs,
        mesh=mesh,
        compiler_params=pltpu.CompilerParams(use_tc_tiling_on_sc=False),
        scratch_types=[
            pltpu.VMEM_SHARED((N_PAD, W), F32),
            pltpu.VMEM((64, W), F32),
            pltpu.VMEM((4, 128), jnp.int32),
            pltpu.VMEM((2, 128), F32),
            pltpu.VMEM((256, W), F32),
            pltpu.SemaphoreType.DMA,
            pltpu.SemaphoreType.DMA,
            pltpu.VMEM((4, 128), jnp.int32),
            pltpu.VMEM((2, 128), F32),
            pltpu.VMEM((256, W), F32),
            pltpu.SemaphoreType.DMA,
            pltpu.SemaphoreType.DMA,
            pltpu.VMEM((64, W), F32),
            pltpu.VMEM((64, W), F32),
        ],
    )


def _gather_body(table, ridx, out, idx_v, rows_v, sem):
    c = lax.axis_index("c")
    s = lax.axis_index("s")
    w = s * 2 + c
    for i in range(13):
        ch = w + i * 32

        @pl.when(ch < 400)
        def _():
            pltpu.sync_copy(ridx.at[ch], idx_v)
            pltpu.async_copy(table.at[idx_v], rows_v, sem).wait()
            pltpu.sync_copy(rows_v, out.at[pl.ds(ch * 128, 128)])


@functools.lru_cache(maxsize=None)
def _make_gather():
    mesh = plsc.VectorSubcoreMesh(core_axis_name="c", subcore_axis_name="s",
                                  num_cores=2, num_subcores=16)
    return pl.kernel(
        _gather_body,
        out_type=[jax.ShapeDtypeStruct((BATCH * SEQ, EMBP), F32)],
        mesh=mesh,
        scratch_types=[
            pltpu.VMEM((128,), jnp.int32),
            pltpu.VMEM((128, EMBP), F32),
            pltpu.SemaphoreType.DMA,
        ],
    )


BB = 128


def _attn_body(seq_ref, zm_ref, mf_ref, sl_ref, pos_ref, w1a_ref, w1b_ref,
               b1_ref, g1W_ref, g1b_ref, g2W_ref, w2_ref, out_ref):
    prec = lax.Precision.HIGHEST
    sh = seq_ref[...] * zm_ref[...][:, :, None]
    hs = jnp.sum(sh, axis=1) / sl_ref[...]
    posp = jnp.dot(pos_ref[...], w1a_ref[...],
                   preferred_element_type=F32, precision=prec)
    t = jnp.dot(sh.reshape(BB * SEQ, EMBP), w1b_ref[...],
                preferred_element_type=F32, precision=prec)
    nh = jnp.tanh(t.reshape(BB, SEQ, EMB) + posp[None] + b1_ref[...])
    g = jnp.dot(nh.reshape(BB * SEQ, EMB), g1W_ref[...],
                preferred_element_type=F32, precision=prec).reshape(BB, SEQ, EMB)
    h2 = jnp.dot(hs, g2W_ref[...],
                 preferred_element_type=F32, precision=prec)
    nh2 = jax.nn.sigmoid(g + g1b_ref[...] + h2[:, None, :])
    beta = jnp.sum(nh2 * w2_ref[...], axis=-1, keepdims=True)
    beta = beta * mf_ref[...][:, :, None]
    sel = jnp.sum(beta * sh, axis=1)
    out_ref[...] = sel[:, :EMB]


def _attn(seqh3, zmask, maskf, slen, pos50, w1a, w1b, b1, g1W, g1b, g2W, w2r):
    grid = (BATCH // BB,)
    full = lambda shape: pl.BlockSpec(shape, lambda b: (0,) * len(shape))
    return pl.pallas_call(
        _attn_body,
        grid=grid,
        in_specs=[
            pl.BlockSpec((BB, SEQ, EMBP), lambda b: (b, 0, 0)),
            pl.BlockSpec((BB, SEQ), lambda b: (b, 0)),
            pl.BlockSpec((BB, SEQ), lambda b: (b, 0)),
            pl.BlockSpec((BB, 1), lambda b: (b, 0)),
            full((SEQ, EMB)),
            full((EMB, EMB)),
            full((EMBP, EMB)),
            full((1, EMB)),
            full((EMB, EMB)),
            full((1, EMB)),
            full((EMBP, EMB)),
            full((1, EMB)),
        ],
        out_specs=pl.BlockSpec((BB, EMB), lambda b: (b, 0)),
        out_shape=jax.ShapeDtypeStruct((BATCH, EMB), F32),
    )(seqh3, zmask, maskf, slen, pos50, w1a, w1b, b1, g1W, g1b, g2W, w2r)


def kernel(embedding, pos_embedding, w1_W, w1_b, w_2, glu1_W, glu1_b, glu2_W,
           adj_val, session_len, adj_idx, session_item, reversed_sess_item,
           mask):
    row = adj_idx[0].astype(jnp.int32)
    col = adj_idx[1].astype(jnp.int32)
    pad_n = NNZ_PAD - NNZ
    fill = (jnp.arange(pad_n, dtype=jnp.int32) * 977) % N_NODE
    colp = jnp.concatenate([col, fill])
    rowp = jnp.concatenate([row, fill])
    valp = jnp.concatenate([adj_val, jnp.zeros((pad_n,), F32)])
    nbat = NNZ_PAD // 256
    coladj = (colp[None, :]
              + (jnp.arange(NCH, dtype=jnp.int32) * N_PAD)[:, None]
              ).reshape(NCH, nbat, 2, 128)
    rowb = jnp.broadcast_to(rowp.reshape(1, nbat, 2, 128),
                            (NCH, nbat, 2, 128))
    icr = jnp.concatenate([coladj, rowb], axis=2)
    valr = valp.reshape(nbat, 2, 128)

    emb_pad = jnp.pad(embedding, ((0, N_PAD - N_NODE), (0, EMBP - EMB)))
    embflat = emb_pad.reshape(N_PAD, NCH, W).transpose(1, 0, 2) \
                     .reshape(NCH * N_PAD, W)

    y1, c1 = _make_spmm(1.0, True)(embflat, icr, valr, embflat)
    comb2, = _make_spmm(1.0 / 3.0, False)(y1, icr, valr, c1)
    tablepad = comb2.reshape(NCH, N_PAD, W).transpose(1, 0, 2) \
                    .reshape(N_PAD, EMBP)
    item_hg = tablepad[:N_NODE, :EMB]

    ridx = jnp.maximum(reversed_sess_item.astype(jnp.int32) - 1, 0) \
              .reshape(BATCH * SEQ // 128, 128)
    seqh, = _make_gather()(tablepad, ridx)
    seqh3 = seqh.reshape(BATCH, SEQ, EMBP)

    zmask = (reversed_sess_item != 0).astype(F32)
    maskf = mask.astype(F32)
    w1a = w1_W[:EMB]
    w1b = jnp.pad(w1_W[EMB:], ((0, EMBP - EMB), (0, 0)))
    g2W = jnp.pad(glu2_W, ((0, EMBP - EMB), (0, 0)))
    select = _attn(seqh3, zmask, maskf, session_len, pos_embedding[:SEQ],
                   w1a, w1b, w1_b.reshape(1, EMB), glu1_W,
                   glu1_b.reshape(1, EMB), g2W, w_2.reshape(1, EMB))
    return (item_hg, select)
